# baseline (speedup 1.0000x reference)
import functools

import jax
import jax.numpy as jnp
from jax.experimental import pallas as pl
from jax.experimental.pallas import tpu as pltpu

_CHARB_EPS2 = 1.0e-6


def _stencil_kernel(w_ref, x_ref, t_ref, o_ref, *, ksize, cpb, m):
    _, H, W = x_ref.shape
    Ho = H - ksize + 1
    Wo = W - ksize + 1
    KK = ksize * ksize
    Q = 48
    LOAD = Q + 8
    starts = list(range(0, Ho - Q, Q)) + [Ho - Q]

    def chan_body(c, tot):
        wv = [[w_ref[c * m + r, tap] for tap in range(KK)]
              for r in range(m)]
        for idx, s in enumerate(starts):
            drop = idx * Q - s
            base = (s // 8) * 8
            off = s - base
            rows = (x_ref[c, base:base + LOAD]
                    - t_ref[c, base:base + LOAD])
            d0 = rows.astype(jnp.bfloat16)
            d1 = pltpu.roll(rows, LOAD - (off + 1), 0)[:Q].astype(jnp.bfloat16)
            d2 = pltpu.roll(rows, LOAD - (off + 2), 0)[:Q].astype(jnp.bfloat16)
            p0 = [d0[:, kj:kj + Wo] for kj in range(ksize)]
            p1 = [d1[:, kj:kj + Wo] for kj in range(ksize)]
            p2 = [d2[:, kj:kj + Wo] for kj in range(ksize)]
            wnd = [None] * (ksize * ksize)
            for kj in range(ksize):
                wnd[0 * ksize + kj] = p0[kj][off:off + Q]
                wnd[1 * ksize + kj] = p1[kj]
                wnd[2 * ksize + kj] = p2[kj]
            for r in range(m):
                acc = wv[r][0] * wnd[0]
                for tap in range(1, KK):
                    acc = acc + wv[r][tap] * wnd[tap]
                y = (acc * acc
                     + jnp.bfloat16(_CHARB_EPS2)).astype(jnp.float32)
                v = y * jax.lax.rsqrt(y)
                if drop:
                    rowid = jax.lax.broadcasted_iota(jnp.int32, v.shape, 0)
                    v = jnp.where(rowid >= drop, v, 0.0)
                tot = tot + v
        return tot

    tot = jax.lax.fori_loop(0, cpb, chan_body,
                            jnp.zeros((Q, Wo), jnp.float32), unroll=8)
    o_ref[...] = jnp.sum(tot, axis=0, keepdims=True)[None]


def kernel(x, t, weight):
    B, C, H, W = x.shape
    OC, _, K, _ = weight.shape
    m = OC // C
    Ho, Wo = H - K + 1, W - K + 1

    x3 = x.reshape(B * C, H, W).astype(jnp.float32)
    t3 = t.reshape(B * C, H, W).astype(jnp.float32)
    w2 = weight[:, 0].astype(jnp.bfloat16).reshape(OC, K * K)

    out = pl.pallas_call(
        functools.partial(_stencil_kernel, ksize=K, cpb=C, m=m),
        grid=(B,),
        in_specs=[
            pl.BlockSpec(memory_space=pltpu.SMEM),
            pl.BlockSpec((C, H, W), lambda b: (b, 0, 0)),
            pl.BlockSpec((C, H, W), lambda b: (b, 0, 0)),
        ],
        out_specs=pl.BlockSpec((1, 1, Wo), lambda b: (b, 0, 0)),
        out_shape=jax.ShapeDtypeStruct((B, 1, Wo), jnp.float32),
        compiler_params=pltpu.CompilerParams(
            dimension_semantics=("parallel",),
        ),
    )(w2, x3, t3)

    denom = float(B * OC * Ho * Wo)
    return jnp.sum(out) / jnp.float32(denom)


# speedup vs baseline: 1.0248x; 1.0248x over previous
import functools

import jax
import jax.numpy as jnp
from jax.experimental import pallas as pl
from jax.experimental.pallas import tpu as pltpu

_CHARB_EPS2 = 1.0e-6


def _stencil_kernel(w_ref, x_ref, t_ref, o_ref, *, ksize, cpb, m):
    _, H, W = x_ref.shape
    Ho = H - ksize + 1
    Wo = W - ksize + 1
    KK = ksize * ksize
    Q = 48
    LOAD = Q + 8
    starts = list(range(0, Ho - Q, Q)) + [Ho - Q]

    def chan_body(c, tot):
        wv = [[w_ref[c * m + r, tap] for tap in range(KK)]
              for r in range(m)]
        for idx, s in enumerate(starts):
            drop = idx * Q - s
            base = (s // 8) * 8
            off = s - base
            rows = (x_ref[c, base:base + LOAD]
                    - t_ref[c, base:base + LOAD])
            d0 = rows.astype(jnp.bfloat16)
            d1 = pltpu.roll(rows, LOAD - (off + 1), 0)[:Q].astype(jnp.bfloat16)
            p0 = [d0[:, kj:kj + Wo] for kj in range(ksize)]
            p1 = [d1[:, kj:kj + Wo] for kj in range(ksize)]
            wnd = [None] * (ksize * ksize)
            for kj in range(ksize):
                wnd[0 * ksize + kj] = p0[kj][off:off + Q]
                wnd[1 * ksize + kj] = p1[kj]
                wnd[2 * ksize + kj] = p0[kj][off + 2:off + 2 + Q]
            for r in range(m):
                acc = wv[r][0] * wnd[0]
                for tap in range(1, KK):
                    acc = acc + wv[r][tap] * wnd[tap]
                y = (acc * acc
                     + jnp.bfloat16(_CHARB_EPS2)).astype(jnp.float32)
                v = y * jax.lax.rsqrt(y)
                if drop:
                    rowid = jax.lax.broadcasted_iota(jnp.int32, v.shape, 0)
                    v = jnp.where(rowid >= drop, v, 0.0)
                tot = tot + v
        return tot

    tot = jax.lax.fori_loop(0, cpb, chan_body,
                            jnp.zeros((Q, Wo), jnp.float32), unroll=16)
    o_ref[...] = jnp.sum(tot, axis=0, keepdims=True)[None]


def kernel(x, t, weight):
    B, C, H, W = x.shape
    OC, _, K, _ = weight.shape
    m = OC // C
    Ho, Wo = H - K + 1, W - K + 1

    x3 = x.reshape(B * C, H, W).astype(jnp.float32)
    t3 = t.reshape(B * C, H, W).astype(jnp.float32)
    w2 = weight[:, 0].astype(jnp.bfloat16).reshape(OC, K * K)

    out = pl.pallas_call(
        functools.partial(_stencil_kernel, ksize=K, cpb=C, m=m),
        grid=(B,),
        in_specs=[
            pl.BlockSpec(memory_space=pltpu.SMEM),
            pl.BlockSpec((C, H, W), lambda b: (b, 0, 0)),
            pl.BlockSpec((C, H, W), lambda b: (b, 0, 0)),
        ],
        out_specs=pl.BlockSpec((1, 1, Wo), lambda b: (b, 0, 0)),
        out_shape=jax.ShapeDtypeStruct((B, 1, Wo), jnp.float32),
        compiler_params=pltpu.CompilerParams(
            dimension_semantics=("parallel",),
        ),
    )(w2, x3, t3)

    denom = float(B * OC * Ho * Wo)
    return jnp.sum(out) / jnp.float32(denom)
